# baseline (speedup 1.0000x reference)
import jax
import jax.numpy as jnp
from jax import lax
from jax.experimental import pallas as pl
from jax.experimental.pallas import tpu as pltpu
from jax.experimental.pallas import tpu_sc as plsc

_ROWS, _DIM = 100000, 64
_CHUNK = 1000
_NCHUNK = _ROWS // _CHUNK
_NW = 32


def _copy_body(x_hbm, o_hbm, buf):
    wid = lax.axis_index("c") * 16 + lax.axis_index("s")

    def do_chunk(ci):
        base = ci * _CHUNK
        pltpu.sync_copy(x_hbm.at[pl.ds(base, _CHUNK)], buf)
        pltpu.sync_copy(buf, o_hbm.at[pl.ds(base, _CHUNK)])

    for g in range(_NCHUNK // _NW):
        do_chunk(wid + g * _NW)

    @pl.when(wid < _NCHUNK % _NW)
    def _():
        do_chunk(wid + (_NCHUNK // _NW) * _NW)


def kernel(code_embeddings):
    k = pl.kernel(
        _copy_body,
        out_type=jax.ShapeDtypeStruct((_ROWS, _DIM), jnp.float32),
        mesh=plsc.VectorSubcoreMesh(core_axis_name="c", subcore_axis_name="s"),
        scratch_types=[pltpu.VMEM((_CHUNK, _DIM), jnp.float32)],
    )
    return k(code_embeddings)
